# Initial kernel scaffold; baseline (speedup 1.0000x reference)
import jax
import jax.numpy as jnp
from jax.experimental import pallas as pl


def kernel(x, edge_index, relations, relation_index, W_l, b_l, W_r, b_r, W_e, att, bias, W_le, b_le):
    raise NotImplementedError("write your pallas kernel here")


# speedup vs baseline: 10.3305x; 10.3305x over previous
import functools

import jax
import jax.numpy as jnp
from jax import lax
from jax.experimental import pallas as pl
from jax.experimental.pallas import tpu as pltpu
from jax.experimental.pallas import tpu_sc as plsc

N = 10000
E = 160000
D = 128
H = 4
R = 64
HD = H * D

NC = 2
NS = 16
NW = NC * NS

C = 64
ET = E + N
CHUNKS_PW = -(-ET // (NW * C))
EPW = CHUNKS_PW * C
EP = EPW * NW
DUMP = N
NACC = 10240
ROWS_PER_TILE = NACC // NS
ETR = 72

CNT_C = 64
CNT_FULL = E // (NW * CNT_C)
CNT_TAIL = (E - NW * CNT_FULL * CNT_C) // CNT_C

_mesh = plsc.VectorSubcoreMesh(core_axis_name="c", subcore_axis_name="s")


@functools.partial(
    pl.kernel,
    out_type=jax.ShapeDtypeStruct((NC, R, 16), jnp.float32),
    mesh=_mesh,
    compiler_params=pltpu.CompilerParams(use_tc_tiling_on_sc=False),
    scratch_types=[
        pltpu.VMEM((CNT_C,), jnp.int32),
        pltpu.VMEM((CNT_C, 16), jnp.float32),
        pltpu.VMEM((R, 16), jnp.float32),
        pltpu.VMEM_SHARED((R, 16), jnp.float32),
    ],
)
def _count_kernel(rel_hbm, out_hbm, relv, ones_v, zb, cnt_sh):
    cid = lax.axis_index("c")
    sid = lax.axis_index("s")
    wid = sid * NC + cid
    lane = lax.iota(jnp.int32, 16)
    zero16 = jnp.zeros((16,), jnp.float32)
    one0 = jnp.where(lane == 0, 1.0, 0.0).astype(jnp.float32)

    def _init(i, _):
        ones_v[i, :] = one0
        return 0

    lax.fori_loop(0, CNT_C, _init, 0)

    def _zrow(i, _):
        zb[i, :] = zero16
        return 0

    lax.fori_loop(0, R, _zrow, 0)

    @pl.when(sid == 0)
    def _():
        pltpu.sync_copy(zb, cnt_sh)

    plsc.subcore_barrier()

    base0 = wid * CNT_FULL * CNT_C

    def _chunk(ci, _):
        b = base0 + ci * CNT_C
        pltpu.sync_copy(rel_hbm.at[pl.ds(b, CNT_C)], relv)
        pltpu.sync_copy(ones_v, cnt_sh.at[relv], add=True)
        return 0

    lax.fori_loop(0, CNT_FULL, _chunk, 0)

    @pl.when(wid < CNT_TAIL)
    def _():
        b = NW * CNT_FULL * CNT_C + wid * CNT_C
        pltpu.sync_copy(rel_hbm.at[pl.ds(b, CNT_C)], relv)
        pltpu.sync_copy(ones_v, cnt_sh.at[relv], add=True)

    plsc.subcore_barrier()

    @pl.when(sid == 0)
    def _():
        pltpu.sync_copy(cnt_sh, out_hbm.at[cid])


def _proj_body(x_ref, wl_ref, bl_ref, wr_ref, br_ref, xl_ref, xr_ref):
    xb = x_ref[...]
    xl_ref[...] = jnp.dot(xb, wl_ref[...], preferred_element_type=jnp.float32) + bl_ref[...]
    xr_ref[...] = jnp.dot(xb, wr_ref[...], preferred_element_type=jnp.float32) + br_ref[...]


_proj_call = pl.pallas_call(
    _proj_body,
    grid=(10,),
    in_specs=[
        pl.BlockSpec((1000, D), lambda i: (i, 0)),
        pl.BlockSpec((D, HD), lambda i: (0, 0)),
        pl.BlockSpec((1, HD), lambda i: (0, 0)),
        pl.BlockSpec((D, HD), lambda i: (0, 0)),
        pl.BlockSpec((1, HD), lambda i: (0, 0)),
    ],
    out_specs=[
        pl.BlockSpec((1000, HD), lambda i: (i, 0)),
        pl.BlockSpec((1000, HD), lambda i: (i, 0)),
    ],
    out_shape=[
        jax.ShapeDtypeStruct((N, HD), jnp.float32),
        jax.ShapeDtypeStruct((N, HD), jnp.float32),
    ],
)


def _etab_body(rel_ref, we_ref, wle_ref, ble_ref, cnt_ref, etab_ref, relout_ref):
    p = jnp.dot(rel_ref[...], we_ref[...], preferred_element_type=jnp.float32)
    etab_ref[0:R, :] = p
    counts = jnp.dot(jnp.ones((1, 2 * NS), jnp.float32), cnt_ref[...],
                     preferred_element_type=jnp.float32)
    e_mean = jnp.dot(counts * (1.0 / E), p, preferred_element_type=jnp.float32)
    etab_ref[R:ETR, :] = jnp.broadcast_to(e_mean, (ETR - R, HD))
    relout_ref[...] = (
        jnp.dot(jnp.maximum(p, 0.0), wle_ref[...], preferred_element_type=jnp.float32)
        + ble_ref[...]
    )


_etab_call = pl.pallas_call(
    _etab_body,
    in_specs=[
        pl.BlockSpec((R, D), lambda: (0, 0)),
        pl.BlockSpec((D, HD), lambda: (0, 0)),
        pl.BlockSpec((HD, D), lambda: (0, 0)),
        pl.BlockSpec((1, D), lambda: (0, 0)),
        pl.BlockSpec((2 * NS, R), lambda: (0, 0)),
    ],
    out_specs=[
        pl.BlockSpec((ETR, HD), lambda: (0, 0)),
        pl.BlockSpec((R, D), lambda: (0, 0)),
    ],
    out_shape=[
        jax.ShapeDtypeStruct((ETR, HD), jnp.float32),
        jax.ShapeDtypeStruct((R, D), jnp.float32),
    ],
)


@functools.partial(
    pl.kernel,
    out_type=(
        jax.ShapeDtypeStruct((EP, 16), jnp.float32),
        jax.ShapeDtypeStruct((NACC, 16), jnp.float32),
        jax.ShapeDtypeStruct((NACC, 16), jnp.float32),
    ),
    mesh=_mesh,
    compiler_params=pltpu.CompilerParams(use_tc_tiling_on_sc=False),
    scratch_types=[
        pltpu.VMEM((C,), jnp.int32),
        pltpu.VMEM((C,), jnp.int32),
        pltpu.VMEM((C,), jnp.int32),
        pltpu.VMEM((C, HD), jnp.float32),
        pltpu.VMEM((C, HD), jnp.float32),
        pltpu.VMEM((C, HD), jnp.float32),
        pltpu.VMEM((C, 16), jnp.float32),
        pltpu.VMEM((HD,), jnp.float32),
        pltpu.VMEM_SHARED((NACC, 16), jnp.float32),
        pltpu.SemaphoreType.DMA,
        pltpu.SemaphoreType.DMA,
        pltpu.SemaphoreType.DMA,
    ],
)
def _alpha_kernel(xl_hbm, xr_hbm, etab_hbm, att_hbm, src_hbm, dst_hbm, rel_hbm,
                  ex_hbm, den0_hbm, den1_hbm,
                  srcv, dstv, relv, xlb, xrb, eb, exb, attb, den_sh,
                  s0, s1, s2):
    cid = lax.axis_index("c")
    sid = lax.axis_index("s")
    wid = sid * NC + cid
    lane = lax.iota(jnp.int32, 16)
    mask4 = lane < H
    ohs = [jnp.where(lane == h, 1.0, 0.0).astype(jnp.float32) for h in range(H)]
    perms = [jnp.bitwise_xor(lane, stp) for stp in (8, 4, 2, 1)]
    zero16 = jnp.zeros((16,), jnp.float32)

    gdn = lax.GatherDimensionNumbers(
        offset_dims=(), collapsed_slice_dims=(0,), start_index_map=(0,))

    def _lanesum(v):
        for p in perms:
            v = v + lax.gather(v, p[:, None], gdn, slice_sizes=(1,),
                               mode=lax.GatherScatterMode.PROMISE_IN_BOUNDS)
        return v

    pltpu.sync_copy(att_hbm, attb)

    def _zr(i, _):
        exb[i, :] = zero16
        return 0

    lax.fori_loop(0, C, _zr, 0)

    def _zcp(i, _):
        pltpu.sync_copy(exb, den_sh.at[pl.ds(sid * ROWS_PER_TILE + i * C, C)])
        return 0

    lax.fori_loop(0, ROWS_PER_TILE // C, _zcp, 0)
    plsc.subcore_barrier()

    base0 = wid * EPW

    def _chunk(ci, _):
        b = base0 + ci * C
        pltpu.sync_copy(src_hbm.at[pl.ds(b, C)], srcv)
        pltpu.sync_copy(dst_hbm.at[pl.ds(b, C)], dstv)
        pltpu.sync_copy(rel_hbm.at[pl.ds(b, C)], relv)
        cp0 = pltpu.async_copy(xl_hbm.at[srcv], xlb, s0)
        cp1 = pltpu.async_copy(xr_hbm.at[dstv], xrb, s1)
        cp2 = pltpu.async_copy(etab_hbm.at[relv], eb, s2)
        cp0.wait()
        cp1.wait()
        cp2.wait()

        def _edge(k, _):
            alpha_vec = zero16
            for h in range(H):
                acc = zero16
                for j in range(8):
                    off = h * D + j * 16
                    s = (xlb[k, pl.ds(off, 16)] + xrb[k, pl.ds(off, 16)]
                         + eb[k, pl.ds(off, 16)])
                    s = jnp.maximum(s, 0.2 * s)
                    acc = acc + s * attb[pl.ds(off, 16)]
                alpha_vec = alpha_vec + _lanesum(acc) * ohs[h]
            exb[k, :] = jnp.where(mask4, jnp.exp(alpha_vec), 0.0)
            return 0

        lax.fori_loop(0, C, _edge, 0)
        pltpu.sync_copy(exb, ex_hbm.at[pl.ds(b, C)])
        pltpu.sync_copy(exb, den_sh.at[dstv], add=True)
        return 0

    lax.fori_loop(0, CHUNKS_PW, _chunk, 0)
    plsc.subcore_barrier()

    @pl.when(cid == 0)
    def _():
        pltpu.sync_copy(den_sh.at[pl.ds(sid * ROWS_PER_TILE, ROWS_PER_TILE)],
                        den0_hbm.at[pl.ds(sid * ROWS_PER_TILE, ROWS_PER_TILE)])

    @pl.when(cid == 1)
    def _():
        pltpu.sync_copy(den_sh.at[pl.ds(sid * ROWS_PER_TILE, ROWS_PER_TILE)],
                        den1_hbm.at[pl.ds(sid * ROWS_PER_TILE, ROWS_PER_TILE)])


@functools.partial(
    pl.kernel,
    out_type=(
        jax.ShapeDtypeStruct((NACC, D), jnp.float32),
        jax.ShapeDtypeStruct((NACC, D), jnp.float32),
    ),
    mesh=_mesh,
    compiler_params=pltpu.CompilerParams(use_tc_tiling_on_sc=False),
    scratch_types=[
        pltpu.VMEM((C,), jnp.int32),
        pltpu.VMEM((C,), jnp.int32),
        pltpu.VMEM((C, HD), jnp.float32),
        pltpu.VMEM((C, 16), jnp.float32),
        pltpu.VMEM((C, 16), jnp.float32),
        pltpu.VMEM((C, 16), jnp.float32),
        pltpu.VMEM((C, D), jnp.float32),
        pltpu.VMEM_SHARED((NACC, D), jnp.float32),
        pltpu.SemaphoreType.DMA,
        pltpu.SemaphoreType.DMA,
        pltpu.SemaphoreType.DMA,
    ],
)
def _aggr_kernel(xl_hbm, ex_hbm, src_hbm, dst_hbm, den0_hbm, den1_hbm,
                 outp0_hbm, outp1_hbm,
                 srcv, dstv, xlb, exb, d0b, d1b, cb, out_sh, s0, s1, s2):
    cid = lax.axis_index("c")
    sid = lax.axis_index("s")
    wid = sid * NC + cid
    zero16 = jnp.zeros((16,), jnp.float32)
    idx_splats = [jnp.full((16, 1), h, jnp.int32) for h in range(H)]
    gdn = lax.GatherDimensionNumbers(
        offset_dims=(), collapsed_slice_dims=(0,), start_index_map=(0,))

    def _zr(i, _):
        for j in range(8):
            cb[i, pl.ds(j * 16, 16)] = zero16
        return 0

    lax.fori_loop(0, C, _zr, 0)

    def _zcp(i, _):
        pltpu.sync_copy(cb, out_sh.at[pl.ds(sid * ROWS_PER_TILE + i * C, C)])
        return 0

    lax.fori_loop(0, ROWS_PER_TILE // C, _zcp, 0)
    plsc.subcore_barrier()

    base0 = wid * EPW

    def _chunk(ci, _):
        b = base0 + ci * C
        pltpu.sync_copy(src_hbm.at[pl.ds(b, C)], srcv)
        pltpu.sync_copy(dst_hbm.at[pl.ds(b, C)], dstv)
        cp0 = pltpu.async_copy(xl_hbm.at[srcv], xlb, s0)
        cp1 = pltpu.async_copy(den0_hbm.at[dstv], d0b, s1)
        cp2 = pltpu.async_copy(den1_hbm.at[dstv], d1b, s2)
        pltpu.sync_copy(ex_hbm.at[pl.ds(b, C)], exb)
        cp0.wait()
        cp1.wait()
        cp2.wait()

        def _edge(k, _):
            w = exb[k, :] / (d0b[k, :] + d1b[k, :] + 1e-16) * 0.25
            whs = [lax.gather(w, idx_splats[h], gdn, slice_sizes=(1,),
                              mode=lax.GatherScatterMode.PROMISE_IN_BOUNDS)
                   for h in range(H)]
            for j in range(8):
                accj = zero16
                for h in range(H):
                    accj = accj + whs[h] * xlb[k, pl.ds(h * D + j * 16, 16)]
                cb[k, pl.ds(j * 16, 16)] = accj
            return 0

        lax.fori_loop(0, C, _edge, 0)
        pltpu.sync_copy(cb, out_sh.at[dstv], add=True)
        return 0

    lax.fori_loop(0, CHUNKS_PW, _chunk, 0)
    plsc.subcore_barrier()

    @pl.when(cid == 0)
    def _():
        pltpu.sync_copy(out_sh.at[pl.ds(sid * ROWS_PER_TILE, ROWS_PER_TILE)],
                        outp0_hbm.at[pl.ds(sid * ROWS_PER_TILE, ROWS_PER_TILE)])

    @pl.when(cid == 1)
    def _():
        pltpu.sync_copy(out_sh.at[pl.ds(sid * ROWS_PER_TILE, ROWS_PER_TILE)],
                        outp1_hbm.at[pl.ds(sid * ROWS_PER_TILE, ROWS_PER_TILE)])


def _comb_body(p0_ref, p1_ref, b_ref, o_ref):
    o_ref[...] = p0_ref[...] + p1_ref[...] + b_ref[...]


_comb_call = pl.pallas_call(
    _comb_body,
    grid=(10,),
    in_specs=[
        pl.BlockSpec((1000, D), lambda i: (i, 0)),
        pl.BlockSpec((1000, D), lambda i: (i, 0)),
        pl.BlockSpec((1, D), lambda i: (0, 0)),
    ],
    out_specs=pl.BlockSpec((1000, D), lambda i: (i, 0)),
    out_shape=jax.ShapeDtypeStruct((N, D), jnp.float32),
)


def kernel(x, edge_index, relations, relation_index, W_l, b_l, W_r, b_r,
           W_e, att, bias, W_le, b_le):
    src = edge_index[0].astype(jnp.int32)
    dst = edge_index[1].astype(jnp.int32)
    rel = relation_index.astype(jnp.int32)
    loop = jnp.arange(N, dtype=jnp.int32)
    npad = EP - ET
    src_f = jnp.concatenate([src, loop, jnp.zeros((npad,), jnp.int32)])
    dst_f = jnp.concatenate([dst, loop, jnp.full((npad,), DUMP, jnp.int32)])
    rel_f = jnp.concatenate([rel, jnp.full((N + npad,), R, jnp.int32)])

    cnt_part = _count_kernel(rel)
    cnt_t = cnt_part.transpose(0, 2, 1).reshape(2 * NS, R)
    x_l, x_r = _proj_call(x, W_l, b_l.reshape(1, HD), W_r, b_r.reshape(1, HD))
    e_tab, rel_out = _etab_call(relations, W_e, W_le, b_le.reshape(1, D), cnt_t)
    ex, den0, den1 = _alpha_kernel(x_l, x_r, e_tab, att.reshape(HD),
                                   src_f, dst_f, rel_f)
    outp0, outp1 = _aggr_kernel(x_l, ex, src_f, dst_f, den0, den1)
    out = _comb_call(outp0, outp1, bias.reshape(1, D))
    return (out, rel_out)
